# Initial kernel scaffold; baseline (speedup 1.0000x reference)
import jax
import jax.numpy as jnp
from jax.experimental import pallas as pl


def kernel(x, edge_index, batch, edge_weight, W_l1, b_l1, W_r1, b_r1, W_e1, att1, bias1, W_l2, b_l2, W_r2, b_r2, W_e2, att2, bias2, fc_W, fc_b):
    raise NotImplementedError("write your pallas kernel here")


# speedup vs baseline: 6.7090x; 6.7090x over previous
import functools

import jax
import jax.numpy as jnp
from jax import lax
from jax.experimental import pallas as pl
from jax.experimental.pallas import tpu as pltpu
from jax.experimental.pallas import tpu_sc as plsc

N = 10000
E = 320000
D_IN = 128
D_H = 32
HEADS = 8
N_GRAPHS = 64

NC = 2
NS = 16
LANES = 16

CH = 80
GRP = CH // LANES
ACC_W = 48
BLK = 1000
NBLK = N // BLK
N_PAD = 10240
RPS = N_PAD // NS


def _sc_edge_body(heads_per_core, edges_per_worker, split_edges_by_core,
                  xl_hbm, xr_hbm, src_hbm, dstr_hbm, w_hbm, attwe_hbm,
                  zrows_hbm, acc_hbm,
                  acc_sh, src_v, dst_v, w_v, rows_l, rows_r, out_b, attwe_v,
                  sem_l, sem_r):
    chunks = edges_per_worker // CH
    cid = lax.axis_index("c")
    sid = lax.axis_index("s")
    wrk = cid * NS + sid if split_edges_by_core else sid
    ebase = wrk * edges_per_worker

    pltpu.sync_copy(src_hbm.at[pl.ds(ebase, edges_per_worker)], src_v)
    pltpu.sync_copy(dstr_hbm.at[wrk], dst_v)
    pltpu.sync_copy(w_hbm.at[pl.ds(ebase, edges_per_worker)], w_v)

    zpad = jnp.zeros((LANES,), jnp.float32)

    def _pad_body(r, carry):
        out_b[r, pl.ds(D_H, LANES)] = zpad
        return carry

    lax.fori_loop(0, CH, _pad_body, 0)

    row_ids = [lax.iota(jnp.int32, LANES) + LANES * g for g in range(GRP)]
    col_den = jnp.full((LANES,), D_H, jnp.int32)

    for hl in range(heads_per_core):
        if split_edges_by_core:
            hg = 0
            out_slot = cid
        else:
            hg = cid * heads_per_core + hl
            out_slot = hg
        pltpu.sync_copy(attwe_hbm.at[hg], attwe_v)
        pltpu.sync_copy(zrows_hbm, acc_sh.at[pl.ds(sid * RPS, RPS)])
        plsc.subcore_barrier()

        def _chunk(k, carry):
            cl = pltpu.async_copy(
                xl_hbm.at[hg].at[src_v.at[pl.ds(k * CH, CH)]], rows_l, sem_l)
            cr = pltpu.async_copy(
                xr_hbm.at[hg].at[dst_v.at[k]], rows_r, sem_r)
            cl.wait()
            cr.wait()
            wvs = [w_v[pl.ds(k * CH + LANES * g, LANES)] for g in range(GRP)]
            accs = [jnp.zeros((LANES,), jnp.float32) for _ in range(GRP)]
            for c in range(D_H):
                att_c = attwe_v[0, c, :]
                we_c = attwe_v[1, c, :]
                colc = jnp.full((LANES,), c, jnp.int32)
                for g in range(GRP):
                    vl = plsc.load_gather(rows_l, [row_ids[g], colc])
                    vr = plsc.load_gather(rows_r, [row_ids[g], colc])
                    z = vl + vr + wvs[g] * we_c
                    m = jnp.where(z > 0.0, z, 0.2 * z)
                    accs[g] = accs[g] + m * att_c
            ps = [jnp.exp(a) for a in accs]
            for g in range(GRP):
                plsc.store_scatter(out_b, [row_ids[g], col_den], ps[g])
            for c in range(D_H):
                colc = jnp.full((LANES,), c, jnp.int32)
                for g in range(GRP):
                    vl = plsc.load_gather(rows_l, [row_ids[g], colc])
                    plsc.store_scatter(out_b, [row_ids[g], colc], ps[g] * vl)
            pltpu.sync_copy(out_b, acc_sh.at[dst_v.at[k]], add=True)
            return carry

        lax.fori_loop(0, chunks, _chunk, 0)
        plsc.subcore_barrier()
        pltpu.sync_copy(acc_sh.at[pl.ds(sid * RPS, RPS)],
                        acc_hbm.at[out_slot].at[pl.ds(sid * RPS, RPS)])


def _sc_edge_pass(heads_per_core, edges_per_worker, split_edges_by_core,
                  n_out_slots):
    chunks = edges_per_worker // CH
    mesh = plsc.VectorSubcoreMesh(core_axis_name="c", subcore_axis_name="s")
    return pl.kernel(
        functools.partial(_sc_edge_body, heads_per_core, edges_per_worker,
                          split_edges_by_core),
        out_type=jax.ShapeDtypeStruct((n_out_slots, N_PAD, ACC_W), jnp.float32),
        mesh=mesh,
        compiler_params=pltpu.CompilerParams(
            needs_layout_passes=False, use_tc_tiling_on_sc=False),
        scratch_types=[
            pltpu.VMEM_SHARED((N_PAD, ACC_W), jnp.float32),
            pltpu.VMEM((edges_per_worker,), jnp.int32),
            pltpu.VMEM((chunks, CH), jnp.int32),
            pltpu.VMEM((edges_per_worker,), jnp.float32),
            pltpu.VMEM((CH, D_H), jnp.float32),
            pltpu.VMEM((CH, D_H), jnp.float32),
            pltpu.VMEM((CH, ACC_W), jnp.float32),
            pltpu.VMEM((2, D_H, LANES), jnp.float32),
            pltpu.SemaphoreType.DMA,
            pltpu.SemaphoreType.DMA,
        ],
    )


def _tc1_body(x_ref, wl_ref, bl_ref, wr_ref, br_ref, xl_ref, xr_ref):
    xb = x_ref[...]
    xl_ref[0] = (jnp.dot(xb, wl_ref[0], preferred_element_type=jnp.float32)
                 + bl_ref[0])
    xr_ref[0] = (jnp.dot(xb, wr_ref[0], preferred_element_type=jnp.float32)
                 + br_ref[0])


def _tc2_body(acc_ref, bias1_ref, wl2_ref, bl2_ref, wr2_ref, br2_ref,
              xl2_ref, xr2_ref):
    al = jnp.zeros((BLK, D_H), jnp.float32)
    ar = jnp.zeros((BLK, D_H), jnp.float32)
    for h in range(HEADS):
        num = acc_ref[h, :, 0:D_H]
        den = acc_ref[h, :, D_H:D_H + 1]
        h1 = jnp.maximum(num / (den + 1e-16) + bias1_ref[h], 0.0)
        al = al + jnp.dot(h1, wl2_ref[h], preferred_element_type=jnp.float32)
        ar = ar + jnp.dot(h1, wr2_ref[h], preferred_element_type=jnp.float32)
    xl2_ref[...] = al + bl2_ref[...]
    xr2_ref[...] = ar + br2_ref[...]


def _tc3_body(acc2_ref, bias2_ref, batch_ref, fcw_ref, fcb_ref, out_ref,
              sums_ref, cnts_ref):
    i = pl.program_id(0)

    @pl.when(i == 0)
    def _init():
        sums_ref[...] = jnp.zeros_like(sums_ref)
        cnts_ref[...] = jnp.zeros_like(cnts_ref)

    num = acc2_ref[0, :, 0:D_H] + acc2_ref[1, :, 0:D_H]
    den = acc2_ref[0, :, D_H:D_H + 1] + acc2_ref[1, :, D_H:D_H + 1]
    feat = jnp.maximum(num / (den + 1e-16) + bias2_ref[...], 0.0)
    b = batch_ref[0, 0, :]
    onehot = (b[:, None] ==
              lax.broadcasted_iota(jnp.int32, (BLK, N_GRAPHS), 1)
              ).astype(jnp.float32)
    sums_ref[...] += lax.dot_general(
        onehot, feat, (((0,), (0,)), ((), ())),
        preferred_element_type=jnp.float32)
    cnts_ref[...] += lax.dot_general(
        onehot, jnp.ones((BLK, D_H), jnp.float32), (((0,), (0,)), ((), ())),
        preferred_element_type=jnp.float32)

    @pl.when(i == pl.num_programs(0) - 1)
    def _fin():
        pooled = sums_ref[...] / jnp.maximum(cnts_ref[...], 1.0)
        sig = 1.0 / (1.0 + jnp.exp(-pooled))
        res = jnp.sum(sig * fcw_ref[...], axis=1)
        out_ref[...] = res[:, None] + fcb_ref[...]


def kernel(x, edge_index, batch, edge_weight, W_l1, b_l1, W_r1, b_r1, W_e1,
           att1, bias1, W_l2, b_l2, W_r2, b_r2, W_e2, att2, bias2, fc_W,
           fc_b):
    src = edge_index[0]
    dstr1 = edge_index[1].reshape(NS, E // (NS * CH), CH)
    dstr2 = edge_index[1].reshape(NC * NS, E // (NC * NS * CH), CH)
    w = edge_weight[:, 0]

    wl1 = W_l1.reshape(D_IN, HEADS, D_H).transpose(1, 0, 2)
    wr1 = W_r1.reshape(D_IN, HEADS, D_H).transpose(1, 0, 2)
    bl1 = b_l1.reshape(HEADS, 1, D_H)
    br1 = b_r1.reshape(HEADS, 1, D_H)
    xl_t, xr_t = pl.pallas_call(
        _tc1_body,
        grid=(HEADS, NBLK),
        in_specs=[
            pl.BlockSpec((BLK, D_IN), lambda h, i: (i, 0)),
            pl.BlockSpec((1, D_IN, D_H), lambda h, i: (h, 0, 0)),
            pl.BlockSpec((1, 1, D_H), lambda h, i: (h, 0, 0)),
            pl.BlockSpec((1, D_IN, D_H), lambda h, i: (h, 0, 0)),
            pl.BlockSpec((1, 1, D_H), lambda h, i: (h, 0, 0)),
        ],
        out_specs=[
            pl.BlockSpec((1, BLK, D_H), lambda h, i: (h, i, 0)),
            pl.BlockSpec((1, BLK, D_H), lambda h, i: (h, i, 0)),
        ],
        out_shape=[
            jax.ShapeDtypeStruct((HEADS, N, D_H), jnp.float32),
            jax.ShapeDtypeStruct((HEADS, N, D_H), jnp.float32),
        ],
    )(x, wl1, bl1, wr1, br1)

    attwe1 = jnp.broadcast_to(
        jnp.stack([att1, W_e1.reshape(HEADS, D_H)], axis=1)[..., None],
        (HEADS, 2, D_H, LANES)).astype(jnp.float32)
    zrows = jnp.zeros((RPS, ACC_W), jnp.float32)
    acc1 = _sc_edge_pass(HEADS // NC, E // NS, False, HEADS)(
        xl_t, xr_t, src, dstr1, w, attwe1, zrows)

    wl2 = W_l2.reshape(HEADS, D_H, D_H)
    wr2 = W_r2.reshape(HEADS, D_H, D_H)
    xl2, xr2 = pl.pallas_call(
        _tc2_body,
        grid=(NBLK,),
        in_specs=[
            pl.BlockSpec((HEADS, BLK, ACC_W), lambda i: (0, i, 0)),
            pl.BlockSpec((HEADS, 1, D_H), lambda i: (0, 0, 0)),
            pl.BlockSpec((HEADS, D_H, D_H), lambda i: (0, 0, 0)),
            pl.BlockSpec((1, D_H), lambda i: (0, 0)),
            pl.BlockSpec((HEADS, D_H, D_H), lambda i: (0, 0, 0)),
            pl.BlockSpec((1, D_H), lambda i: (0, 0)),
        ],
        out_specs=[
            pl.BlockSpec((BLK, D_H), lambda i: (i, 0)),
            pl.BlockSpec((BLK, D_H), lambda i: (i, 0)),
        ],
        out_shape=[
            jax.ShapeDtypeStruct((N, D_H), jnp.float32),
            jax.ShapeDtypeStruct((N, D_H), jnp.float32),
        ],
    )(acc1, bias1.reshape(HEADS, 1, D_H), wl2, b_l2.reshape(1, D_H), wr2,
      b_r2.reshape(1, D_H))

    attwe2 = jnp.broadcast_to(
        jnp.stack([att2, W_e2], axis=1)[..., None],
        (1, 2, D_H, LANES)).astype(jnp.float32)
    acc2 = _sc_edge_pass(1, E // (NC * NS), True, NC)(
        xl2.reshape(1, N, D_H), xr2.reshape(1, N, D_H), src, dstr2, w,
        attwe2, zrows)

    out = pl.pallas_call(
        _tc3_body,
        grid=(NBLK,),
        in_specs=[
            pl.BlockSpec((NC, BLK, ACC_W), lambda i: (0, i, 0)),
            pl.BlockSpec((1, D_H), lambda i: (0, 0)),
            pl.BlockSpec((1, 1, BLK), lambda i: (i, 0, 0)),
            pl.BlockSpec((1, D_H), lambda i: (0, 0)),
            pl.BlockSpec((1, 1), lambda i: (0, 0)),
        ],
        out_specs=pl.BlockSpec((N_GRAPHS, 1), lambda i: (0, 0)),
        out_shape=jax.ShapeDtypeStruct((N_GRAPHS, 1), jnp.float32),
        scratch_shapes=[
            pltpu.VMEM((N_GRAPHS, D_H), jnp.float32),
            pltpu.VMEM((N_GRAPHS, D_H), jnp.float32),
        ],
    )(acc2, bias2.reshape(1, D_H), batch.reshape(NBLK, 1, BLK),
      fc_W.reshape(1, D_H), fc_b.reshape(1, 1))
    return out


# speedup vs baseline: 7.6692x; 1.1431x over previous
import functools

import jax
import jax.numpy as jnp
from jax import lax
from jax.experimental import pallas as pl
from jax.experimental.pallas import tpu as pltpu
from jax.experimental.pallas import tpu_sc as plsc

N = 10000
E = 320000
D_IN = 128
D_H = 32
HEADS = 8
N_GRAPHS = 64

NC = 2
NS = 16
LANES = 16

CH = 80
GRP = CH // LANES
ACC_W = 48
BLK = 1000
NBLK = N // BLK
N_PAD = 10240
RPS = N_PAD // NS


def _sc_edge_body(heads_per_core, edges_per_worker, split_edges_by_core,
                  xl_hbm, xr_hbm, src_hbm, dstr_hbm, w_hbm, attwe_hbm,
                  zrows_hbm, acc_hbm,
                  acc_sh, src_v, dst_v, w_v, rows_l, rows_r, out_b, attwe_v,
                  sem_l0, sem_l1, sem_r0, sem_r1, sem_s0, sem_s1):
    chunks = edges_per_worker // CH
    cid = lax.axis_index("c")
    sid = lax.axis_index("s")
    wrk = cid * NS + sid if split_edges_by_core else sid
    ebase = wrk * edges_per_worker

    pltpu.sync_copy(src_hbm.at[pl.ds(ebase, edges_per_worker)], src_v)
    pltpu.sync_copy(dstr_hbm.at[wrk], dst_v)
    pltpu.sync_copy(w_hbm.at[pl.ds(ebase, edges_per_worker)], w_v)

    zpad = jnp.zeros((LANES,), jnp.float32)

    def _pad_body(r, carry):
        out_b[r, pl.ds(D_H, LANES)] = zpad
        return carry

    lax.fori_loop(0, 2 * CH, _pad_body, 0)

    row_ids = [lax.iota(jnp.int32, LANES) + LANES * g for g in range(GRP)]
    col_den = jnp.full((LANES,), D_H, jnp.int32)
    sems_l = (sem_l0, sem_l1)
    sems_r = (sem_r0, sem_r1)
    sems_s = (sem_s0, sem_s1)

    for hl in range(heads_per_core):
        if split_edges_by_core:
            hg = 0
            out_slot = cid
        else:
            hg = cid * heads_per_core + hl
            out_slot = hg
        pltpu.sync_copy(attwe_hbm.at[hg], attwe_v)
        pltpu.sync_copy(zrows_hbm, acc_sh.at[pl.ds(sid * RPS, RPS)])
        plsc.subcore_barrier()

        def _gather_desc(par, k):
            cl = pltpu.make_async_copy(
                xl_hbm.at[hg].at[src_v.at[pl.ds(k * CH, CH)]],
                rows_l.at[pl.ds(par * CH, CH)], sems_l[par])
            cr = pltpu.make_async_copy(
                xr_hbm.at[hg].at[dst_v.at[k]],
                rows_r.at[pl.ds(par * CH, CH)], sems_r[par])
            return cl, cr

        def _scatter_desc(par, k):
            return pltpu.make_async_copy(
                out_b.at[pl.ds(par * CH, CH)],
                acc_sh.at[dst_v.at[k]], sems_s[par])

        for d in _gather_desc(0, 0):
            d.start()

        def _chunk(k, carry):
            par = k & 1
            even = par == 0
            nxt = k + 1
            @pl.when((nxt < chunks) & even)
            def _():
                for d in _gather_desc(1, nxt):
                    d.start()

            @pl.when((nxt < chunks) & jnp.logical_not(even))
            def _():
                for d in _gather_desc(0, nxt):
                    d.start()

            @pl.when(even)
            def _():
                for d in _gather_desc(0, k):
                    d.wait()

            @pl.when(jnp.logical_not(even))
            def _():
                for d in _gather_desc(1, k):
                    d.wait()

            base = par * CH
            rids = [r + base for r in row_ids]
            wvs = [w_v[pl.ds(k * CH + LANES * g, LANES)] for g in range(GRP)]
            accs = [jnp.zeros((LANES,), jnp.float32) for _ in range(GRP)]
            for c in range(D_H):
                att_c = attwe_v[0, c, :]
                we_c = attwe_v[1, c, :]
                colc = jnp.full((LANES,), c, jnp.int32)
                for g in range(GRP):
                    vl = plsc.load_gather(rows_l, [rids[g], colc])
                    vr = plsc.load_gather(rows_r, [rids[g], colc])
                    z = vl + vr + wvs[g] * we_c
                    m = jnp.where(z > 0.0, z, 0.2 * z)
                    accs[g] = accs[g] + m * att_c
            ps = [jnp.exp(a) for a in accs]
            for g in range(GRP):
                plsc.store_scatter(out_b, [rids[g], col_den], ps[g])
            for c in range(D_H):
                colc = jnp.full((LANES,), c, jnp.int32)
                for g in range(GRP):
                    vl = plsc.load_gather(rows_l, [rids[g], colc])
                    plsc.store_scatter(out_b, [rids[g], colc], ps[g] * vl)
            pltpu.sync_copy(out_b.at[pl.ds(base, CH)], acc_sh.at[dst_v.at[k]],
                            add=True)
            return carry

        lax.fori_loop(0, chunks, _chunk, 0)
        plsc.subcore_barrier()
        pltpu.sync_copy(acc_sh.at[pl.ds(sid * RPS, RPS)],
                        acc_hbm.at[out_slot].at[pl.ds(sid * RPS, RPS)])


def _sc_edge_pass(heads_per_core, edges_per_worker, split_edges_by_core,
                  n_out_slots):
    chunks = edges_per_worker // CH
    mesh = plsc.VectorSubcoreMesh(core_axis_name="c", subcore_axis_name="s")
    return pl.kernel(
        functools.partial(_sc_edge_body, heads_per_core, edges_per_worker,
                          split_edges_by_core),
        out_type=jax.ShapeDtypeStruct((n_out_slots, N_PAD, ACC_W), jnp.float32),
        mesh=mesh,
        compiler_params=pltpu.CompilerParams(
            needs_layout_passes=False, use_tc_tiling_on_sc=False),
        scratch_types=[
            pltpu.VMEM_SHARED((N_PAD, ACC_W), jnp.float32),
            pltpu.VMEM((edges_per_worker,), jnp.int32),
            pltpu.VMEM((chunks, CH), jnp.int32),
            pltpu.VMEM((edges_per_worker,), jnp.float32),
            pltpu.VMEM((2 * CH, D_H), jnp.float32),
            pltpu.VMEM((2 * CH, D_H), jnp.float32),
            pltpu.VMEM((2 * CH, ACC_W), jnp.float32),
            pltpu.VMEM((2, D_H, LANES), jnp.float32),
            pltpu.SemaphoreType.DMA,
            pltpu.SemaphoreType.DMA,
            pltpu.SemaphoreType.DMA,
            pltpu.SemaphoreType.DMA,
            pltpu.SemaphoreType.DMA,
            pltpu.SemaphoreType.DMA,
        ],
    )


def _tc1_body(x_ref, wl_ref, bl_ref, wr_ref, br_ref, xl_ref, xr_ref):
    xb = x_ref[...]
    xl_ref[0] = (jnp.dot(xb, wl_ref[0], preferred_element_type=jnp.float32)
                 + bl_ref[0])
    xr_ref[0] = (jnp.dot(xb, wr_ref[0], preferred_element_type=jnp.float32)
                 + br_ref[0])


def _tc2_body(acc_ref, bias1_ref, wl2_ref, bl2_ref, wr2_ref, br2_ref,
              xl2_ref, xr2_ref):
    al = jnp.zeros((BLK, D_H), jnp.float32)
    ar = jnp.zeros((BLK, D_H), jnp.float32)
    for h in range(HEADS):
        num = acc_ref[h, :, 0:D_H]
        den = acc_ref[h, :, D_H:D_H + 1]
        h1 = jnp.maximum(num / (den + 1e-16) + bias1_ref[h], 0.0)
        al = al + jnp.dot(h1, wl2_ref[h], preferred_element_type=jnp.float32)
        ar = ar + jnp.dot(h1, wr2_ref[h], preferred_element_type=jnp.float32)
    xl2_ref[...] = al + bl2_ref[...]
    xr2_ref[...] = ar + br2_ref[...]


def _tc3_body(acc2_ref, bias2_ref, batch_ref, fcw_ref, fcb_ref, out_ref,
              sums_ref, cnts_ref):
    i = pl.program_id(0)

    @pl.when(i == 0)
    def _init():
        sums_ref[...] = jnp.zeros_like(sums_ref)
        cnts_ref[...] = jnp.zeros_like(cnts_ref)

    num = acc2_ref[0, :, 0:D_H] + acc2_ref[1, :, 0:D_H]
    den = acc2_ref[0, :, D_H:D_H + 1] + acc2_ref[1, :, D_H:D_H + 1]
    feat = jnp.maximum(num / (den + 1e-16) + bias2_ref[...], 0.0)
    b = batch_ref[0, 0, :]
    onehot = (b[:, None] ==
              lax.broadcasted_iota(jnp.int32, (BLK, N_GRAPHS), 1)
              ).astype(jnp.float32)
    sums_ref[...] += lax.dot_general(
        onehot, feat, (((0,), (0,)), ((), ())),
        preferred_element_type=jnp.float32)
    cnts_ref[...] += lax.dot_general(
        onehot, jnp.ones((BLK, D_H), jnp.float32), (((0,), (0,)), ((), ())),
        preferred_element_type=jnp.float32)

    @pl.when(i == pl.num_programs(0) - 1)
    def _fin():
        pooled = sums_ref[...] / jnp.maximum(cnts_ref[...], 1.0)
        sig = 1.0 / (1.0 + jnp.exp(-pooled))
        res = jnp.sum(sig * fcw_ref[...], axis=1)
        out_ref[...] = res[:, None] + fcb_ref[...]


def kernel(x, edge_index, batch, edge_weight, W_l1, b_l1, W_r1, b_r1, W_e1,
           att1, bias1, W_l2, b_l2, W_r2, b_r2, W_e2, att2, bias2, fc_W,
           fc_b):
    src = edge_index[0]
    dstr1 = edge_index[1].reshape(NS, E // (NS * CH), CH)
    dstr2 = edge_index[1].reshape(NC * NS, E // (NC * NS * CH), CH)
    w = edge_weight[:, 0]

    wl1 = W_l1.reshape(D_IN, HEADS, D_H).transpose(1, 0, 2)
    wr1 = W_r1.reshape(D_IN, HEADS, D_H).transpose(1, 0, 2)
    bl1 = b_l1.reshape(HEADS, 1, D_H)
    br1 = b_r1.reshape(HEADS, 1, D_H)
    xl_t, xr_t = pl.pallas_call(
        _tc1_body,
        grid=(HEADS, NBLK),
        in_specs=[
            pl.BlockSpec((BLK, D_IN), lambda h, i: (i, 0)),
            pl.BlockSpec((1, D_IN, D_H), lambda h, i: (h, 0, 0)),
            pl.BlockSpec((1, 1, D_H), lambda h, i: (h, 0, 0)),
            pl.BlockSpec((1, D_IN, D_H), lambda h, i: (h, 0, 0)),
            pl.BlockSpec((1, 1, D_H), lambda h, i: (h, 0, 0)),
        ],
        out_specs=[
            pl.BlockSpec((1, BLK, D_H), lambda h, i: (h, i, 0)),
            pl.BlockSpec((1, BLK, D_H), lambda h, i: (h, i, 0)),
        ],
        out_shape=[
            jax.ShapeDtypeStruct((HEADS, N, D_H), jnp.float32),
            jax.ShapeDtypeStruct((HEADS, N, D_H), jnp.float32),
        ],
    )(x, wl1, bl1, wr1, br1)

    attwe1 = jnp.broadcast_to(
        jnp.stack([att1, W_e1.reshape(HEADS, D_H)], axis=1)[..., None],
        (HEADS, 2, D_H, LANES)).astype(jnp.float32)
    zrows = jnp.zeros((RPS, ACC_W), jnp.float32)
    acc1 = _sc_edge_pass(HEADS // NC, E // NS, False, HEADS)(
        xl_t, xr_t, src, dstr1, w, attwe1, zrows)

    wl2 = W_l2.reshape(HEADS, D_H, D_H)
    wr2 = W_r2.reshape(HEADS, D_H, D_H)
    xl2, xr2 = pl.pallas_call(
        _tc2_body,
        grid=(NBLK,),
        in_specs=[
            pl.BlockSpec((HEADS, BLK, ACC_W), lambda i: (0, i, 0)),
            pl.BlockSpec((HEADS, 1, D_H), lambda i: (0, 0, 0)),
            pl.BlockSpec((HEADS, D_H, D_H), lambda i: (0, 0, 0)),
            pl.BlockSpec((1, D_H), lambda i: (0, 0)),
            pl.BlockSpec((HEADS, D_H, D_H), lambda i: (0, 0, 0)),
            pl.BlockSpec((1, D_H), lambda i: (0, 0)),
        ],
        out_specs=[
            pl.BlockSpec((BLK, D_H), lambda i: (i, 0)),
            pl.BlockSpec((BLK, D_H), lambda i: (i, 0)),
        ],
        out_shape=[
            jax.ShapeDtypeStruct((N, D_H), jnp.float32),
            jax.ShapeDtypeStruct((N, D_H), jnp.float32),
        ],
    )(acc1, bias1.reshape(HEADS, 1, D_H), wl2, b_l2.reshape(1, D_H), wr2,
      b_r2.reshape(1, D_H))

    attwe2 = jnp.broadcast_to(
        jnp.stack([att2, W_e2], axis=1)[..., None],
        (1, 2, D_H, LANES)).astype(jnp.float32)
    acc2 = _sc_edge_pass(1, E // (NC * NS), True, NC)(
        xl2.reshape(1, N, D_H), xr2.reshape(1, N, D_H), src, dstr2, w,
        attwe2, zrows)

    out = pl.pallas_call(
        _tc3_body,
        grid=(NBLK,),
        in_specs=[
            pl.BlockSpec((NC, BLK, ACC_W), lambda i: (0, i, 0)),
            pl.BlockSpec((1, D_H), lambda i: (0, 0)),
            pl.BlockSpec((1, 1, BLK), lambda i: (i, 0, 0)),
            pl.BlockSpec((1, D_H), lambda i: (0, 0)),
            pl.BlockSpec((1, 1), lambda i: (0, 0)),
        ],
        out_specs=pl.BlockSpec((N_GRAPHS, 1), lambda i: (0, 0)),
        out_shape=jax.ShapeDtypeStruct((N_GRAPHS, 1), jnp.float32),
        scratch_shapes=[
            pltpu.VMEM((N_GRAPHS, D_H), jnp.float32),
            pltpu.VMEM((N_GRAPHS, D_H), jnp.float32),
        ],
    )(acc2, bias2.reshape(1, D_H), batch.reshape(NBLK, 1, BLK),
      fc_W.reshape(1, D_H), fc_b.reshape(1, 1))
    return out
